# Initial kernel scaffold; baseline (speedup 1.0000x reference)
import jax
import jax.numpy as jnp
from jax.experimental import pallas as pl


def kernel(input_ids, position_ids, segment_ids, age_ids, word_table, pos_table, seg_table, age_table, ln_gamma, ln_beta):
    raise NotImplementedError("write your pallas kernel here")


# speedup vs baseline: 5.0249x; 5.0249x over previous
import functools

import jax
import jax.numpy as jnp
from jax import lax
from jax.experimental import pallas as pl
from jax.experimental.pallas import tpu as pltpu
from jax.experimental.pallas import tpu_sc as plsc

HIDDEN = 128
LANES = 16
NJ = HIDDEN // LANES
EPS = 1e-5
NUM_CORES = 2
NUM_SUBCORES = 16
NUM_WORKERS = NUM_CORES * NUM_SUBCORES
CHUNK = 128


@functools.lru_cache(maxsize=None)
def _build(n_tokens: int, vocab: int, small_rows: int, pos_rows: int,
           seg_rows: int, n_chunks_ceil: int):
  n_per_w = n_tokens // NUM_WORKERS
  n_chunks = n_per_w // CHUNK
  assert n_per_w % CHUNK == 0 and n_tokens % NUM_WORKERS == 0
  seg_base = pos_rows * HIDDEN
  age_base = (pos_rows + seg_rows) * HIDDEN

  mesh = plsc.VectorSubcoreMesh(
      core_axis_name="c", subcore_axis_name="s",
      num_cores=NUM_CORES, num_subcores=NUM_SUBCORES)

  @functools.partial(
      pl.kernel,
      mesh=mesh,
      compiler_params=pltpu.CompilerParams(needs_layout_passes=False),
      out_type=jax.ShapeDtypeStruct((n_tokens, HIDDEN), jnp.float32),
      scratch_types=[
          pltpu.VMEM((CHUNK,), jnp.int32),
          pltpu.VMEM((CHUNK,), jnp.int32),
          pltpu.VMEM((CHUNK,), jnp.int32),
          pltpu.VMEM((CHUNK,), jnp.int32),
          pltpu.VMEM((CHUNK, HIDDEN), jnp.float32),
          pltpu.VMEM((small_rows * HIDDEN,), jnp.float32),
          pltpu.VMEM((2 * HIDDEN,), jnp.float32),
          pltpu.SemaphoreType.DMA,
      ],
  )
  def k(iw_hbm, ip_hbm, is_hbm, ia_hbm, word_hbm, small_hbm, gb_hbm,
        out_hbm, iw_v, ip_v, is_v, ia_v, rows_v, small_v, gb_v, sem):
    wid = lax.axis_index("s") * NUM_CORES + lax.axis_index("c")
    base_w = wid * n_per_w

    pltpu.sync_copy(small_hbm, small_v)
    pltpu.sync_copy(gb_hbm, gb_v)

    iota = lax.iota(jnp.int32, 16)
    gammas = [gb_v[pl.ds(j * LANES, LANES)] for j in range(NJ)]
    betas = [gb_v[pl.ds(HIDDEN + j * LANES, LANES)] for j in range(NJ)]

    def chunk_body(ci, carry):
      base = base_w + ci * CHUNK
      pltpu.sync_copy(iw_hbm.at[pl.ds(base, CHUNK)], iw_v)
      pltpu.sync_copy(ip_hbm.at[pl.ds(base, CHUNK)], ip_v)
      pltpu.sync_copy(is_hbm.at[pl.ds(base, CHUNK)], is_v)
      pltpu.sync_copy(ia_hbm.at[pl.ds(base, CHUNK)], ia_v)
      pltpu.async_copy(word_hbm.at[iw_v], rows_v, sem).wait()

      def group_body(g, gcarry):
        r0 = g * LANES
        pb_v = ip_v[pl.ds(r0, LANES)] * HIDDEN
        sb_v = seg_base + is_v[pl.ds(r0, LANES)] * HIDDEN
        ab_v = age_base + ia_v[pl.ds(r0, LANES)] * HIDDEN
        for t in range(LANES):
          r = r0 + t
          pbase = pb_v[t]
          sbase = sb_v[t]
          abase = ab_v[t]
          xs = []
          sum_v = None
          sumsq_v = None
          for j in range(NJ):
            off = j * LANES
            w = rows_v[r, pl.ds(off, LANES)]
            p = plsc.load_gather(small_v, [pbase + off + iota])
            s = plsc.load_gather(small_v, [sbase + off + iota])
            a = plsc.load_gather(small_v, [abase + off + iota])
            x = w + p + s + a
            xs.append(x)
            sum_v = x if sum_v is None else sum_v + x
            sumsq_v = x * x if sumsq_v is None else sumsq_v + x * x
          mean_s = jnp.sum(sum_v) * (1.0 / HIDDEN)
          var_s = jnp.sum(sumsq_v) * (1.0 / HIDDEN) - mean_s * mean_s
          v_v = jnp.full((LANES,), var_s + EPS, jnp.float32)
          yi = jnp.int32(0x5F3759DF) - (plsc.bitcast(v_v, jnp.int32) >> 1)
          y = plsc.bitcast(yi, jnp.float32)
          half_v = v_v * 0.5
          for _ in range(3):
            y = y * (1.5 - half_v * y * y)
          mean_v = jnp.full((LANES,), mean_s, jnp.float32)
          for j in range(NJ):
            xh = (xs[j] - mean_v) * y
            rows_v[r, pl.ds(j * LANES, LANES)] = xh * gammas[j] + betas[j]
        return gcarry

      lax.fori_loop(0, CHUNK // LANES, group_body, 0)
      pltpu.sync_copy(rows_v, out_hbm.at[pl.ds(base, CHUNK)])
      return carry

    lax.fori_loop(0, n_chunks, chunk_body, 0)

  return k


def kernel(input_ids, position_ids, segment_ids, age_ids, word_table,
           pos_table, seg_table, age_table, ln_gamma, ln_beta):
  b, s = input_ids.shape
  n_tokens = b * s
  iw = input_ids.reshape(-1).astype(jnp.int32)
  ip = position_ids.reshape(-1).astype(jnp.int32)
  iseg = segment_ids.reshape(-1).astype(jnp.int32)
  ia = age_ids.reshape(-1).astype(jnp.int32)
  small = jnp.concatenate(
      [pos_table, seg_table, age_table], axis=0).reshape(-1)
  gb = jnp.concatenate([ln_gamma, ln_beta], axis=0)
  pos_rows = pos_table.shape[0]
  seg_rows = seg_table.shape[0]
  small_rows = pos_rows + seg_rows + age_table.shape[0]
  k = _build(n_tokens, word_table.shape[0], small_rows, pos_rows, seg_rows, 0)
  out = k(iw, ip, iseg, ia, word_table, small, gb)
  return out.reshape(b, s, HIDDEN)
